# Initial kernel scaffold; baseline (speedup 1.0000x reference)
import functools

import jax
import jax.numpy as jnp
from jax import lax
from jax.experimental import pallas as pl
from jax.experimental.pallas import tpu as pltpu
from jax.experimental.pallas import tpu_sc as plsc

BATCH = 16384
HIST = 50
DIM = 32
TOTAL = BATCH * HIST
NUM_CORES = 2
NUM_SUBCORES = 16
NW = NUM_CORES * NUM_SUBCORES
PER_W = TOTAL // NW
CHUNK = 2560
NCHUNK = PER_W // CHUNK
IDX_ROWS = CHUNK // 128

_mesh = plsc.VectorSubcoreMesh(core_axis_name="c", subcore_axis_name="s")


@functools.partial(
    pl.kernel,
    mesh=_mesh,
    out_type=jax.ShapeDtypeStruct((TOTAL, DIM), jnp.float32),
    scratch_types=[
        pltpu.VMEM((IDX_ROWS, 128), jnp.int32),
        pltpu.VMEM((CHUNK, DIM), jnp.float32),
        pltpu.SemaphoreType.DMA,
    ],
)
def _gather_kernel(idx_hbm, table_hbm, out_hbm, idx_v, rows_v, sem):
    wid = lax.axis_index("s") * NUM_CORES + lax.axis_index("c")
    base = wid * PER_W

    def body(g, carry):
        row0 = base + g * CHUNK
        pltpu.sync_copy(idx_hbm.at[pl.ds(row0 // 128, IDX_ROWS)], idx_v)
        copies = [
            pltpu.async_copy(
                table_hbm.at[idx_v.at[j]],
                rows_v.at[pl.ds(j * 128, 128)],
                sem,
            )
            for j in range(IDX_ROWS)
        ]
        for c in copies:
            c.wait()
        pltpu.sync_copy(rows_v, out_hbm.at[pl.ds(row0, CHUNK)])
        return carry

    lax.fori_loop(0, NCHUNK, body, 0)


def kernel(input, weight):
    idx = input.astype(jnp.int32).reshape(TOTAL // 128, 128)
    out = _gather_kernel(idx, weight)
    return out.reshape(BATCH, HIST, DIM)


# speedup vs baseline: 1.0943x; 1.0943x over previous
import functools

import jax
import jax.numpy as jnp
from jax import lax
from jax.experimental import pallas as pl
from jax.experimental.pallas import tpu as pltpu
from jax.experimental.pallas import tpu_sc as plsc

BATCH = 16384
HIST = 50
DIM = 32
TOTAL = BATCH * HIST
NUM_CORES = 2
NUM_SUBCORES = 16
NW = NUM_CORES * NUM_SUBCORES
PER_W = TOTAL // NW
CHUNK = 1024
NCHUNK = PER_W // CHUNK
IDX_ROWS = CHUNK // 128

_mesh = plsc.VectorSubcoreMesh(core_axis_name="c", subcore_axis_name="s")


@functools.partial(
    pl.kernel,
    mesh=_mesh,
    out_type=jax.ShapeDtypeStruct((TOTAL, DIM), jnp.float32),
    scratch_types=[
        pltpu.VMEM((IDX_ROWS, 128), jnp.int32),
        pltpu.VMEM((CHUNK, DIM), jnp.float32),
        pltpu.SemaphoreType.DMA,
    ],
    compiler_params=pltpu.CompilerParams(use_tc_tiling_on_sc=False),
)
def _gather_kernel(idx_hbm, table_hbm, out_hbm, idx_v, rows_v, sem):
    wid = lax.axis_index("s") * NUM_CORES + lax.axis_index("c")
    base = wid * PER_W

    def body(g, carry):
        row0 = pl.multiple_of(base + g * CHUNK, CHUNK)
        idx_row0 = pl.multiple_of(row0 // 128, IDX_ROWS)
        pltpu.sync_copy(idx_hbm.at[pl.ds(idx_row0, IDX_ROWS)], idx_v)
        copies = [
            pltpu.async_copy(
                table_hbm.at[idx_v.at[j]],
                rows_v.at[pl.ds(j * 128, 128)],
                sem,
            )
            for j in range(IDX_ROWS)
        ]
        for c in copies:
            c.wait()
        pltpu.sync_copy(rows_v, out_hbm.at[pl.ds(row0, CHUNK)])
        return carry

    lax.fori_loop(0, NCHUNK, body, 0)


def kernel(input, weight):
    idx = input.astype(jnp.int32).reshape(TOTAL // 128, 128)
    out = _gather_kernel(idx, weight)
    return out.reshape(BATCH, HIST, DIM)


# speedup vs baseline: 1.1100x; 1.0143x over previous
import functools

import jax
import jax.numpy as jnp
from jax import lax
from jax.experimental import pallas as pl
from jax.experimental.pallas import tpu as pltpu
from jax.experimental.pallas import tpu_sc as plsc

BATCH = 16384
HIST = 50
DIM = 32
TOTAL = BATCH * HIST
NUM_CORES = 2
NUM_SUBCORES = 16
NW = NUM_CORES * NUM_SUBCORES
PER_W = TOTAL // NW
IDX_ALL = PER_W // 128
CHUNK = 1280
NCHUNK = PER_W // CHUNK
IDX_ROWS = CHUNK // 128
NBODY = NCHUNK // 2

_mesh = plsc.VectorSubcoreMesh(core_axis_name="c", subcore_axis_name="s")


@functools.partial(
    pl.kernel,
    mesh=_mesh,
    out_type=jax.ShapeDtypeStruct((TOTAL, DIM), jnp.float32),
    scratch_types=[
        pltpu.VMEM((IDX_ALL, 128), jnp.int32),
        pltpu.VMEM((2, CHUNK, DIM), jnp.float32),
        pltpu.SemaphoreType.DMA,
        pltpu.SemaphoreType.DMA,
        pltpu.SemaphoreType.DMA,
        pltpu.SemaphoreType.DMA,
    ],
    compiler_params=pltpu.CompilerParams(use_tc_tiling_on_sc=False),
)
def _gather_kernel(idx_hbm, table_hbm, out_hbm, idx_v, rows_v, sg0, sg1, so0, so1):
    wid = lax.axis_index("s") * NUM_CORES + lax.axis_index("c")
    base = pl.multiple_of(wid * PER_W, PER_W)
    idx_base = pl.multiple_of(wid * IDX_ALL, 8)

    pltpu.sync_copy(idx_hbm.at[pl.ds(idx_base, IDX_ALL)], idx_v)

    def issue_gather(g, buf, sem):
        for j in range(IDX_ROWS):
            pltpu.async_copy(
                table_hbm.at[idx_v.at[g * IDX_ROWS + j]],
                rows_v.at[buf].at[pl.ds(j * 128, 128)],
                sem,
            )

    def wait_gather(g, buf, sem):
        for j in range(IDX_ROWS):
            pltpu.make_async_copy(
                table_hbm.at[idx_v.at[g * IDX_ROWS + j]],
                rows_v.at[buf].at[pl.ds(j * 128, 128)],
                sem,
            ).wait()

    def out_slice(g):
        return out_hbm.at[pl.ds(pl.multiple_of(base + g * CHUNK, CHUNK), CHUNK)]

    issue_gather(0, 0, sg0)

    def body(i, carry):
        g0 = 2 * i
        g1 = 2 * i + 1
        wait_gather(g0, 0, sg0)

        @pl.when(i > 0)
        def _():
            pltpu.make_async_copy(rows_v.at[1], out_slice(g0 - 1), so1).wait()

        issue_gather(g1, 1, sg1)
        pltpu.async_copy(rows_v.at[0], out_slice(g0), so0)
        wait_gather(g1, 1, sg1)
        pltpu.make_async_copy(rows_v.at[0], out_slice(g0), so0).wait()

        @pl.when(i < NBODY - 1)
        def _():
            issue_gather(g1 + 1, 0, sg0)

        pltpu.async_copy(rows_v.at[1], out_slice(g1), so1)
        return carry

    lax.fori_loop(0, NBODY, body, 0)
    pltpu.make_async_copy(rows_v.at[1], out_slice(NCHUNK - 1), so1).wait()


def kernel(input, weight):
    idx = input.astype(jnp.int32).reshape(TOTAL // 128, 128)
    out = _gather_kernel(idx, weight)
    return out.reshape(BATCH, HIST, DIM)


# speedup vs baseline: 1.4918x; 1.3439x over previous
import functools

import jax
import jax.numpy as jnp
from jax import lax
from jax.experimental import pallas as pl
from jax.experimental.pallas import tpu as pltpu
from jax.experimental.pallas import tpu_sc as plsc

BATCH = 16384
HIST = 50
DIM = 32
TOTAL = BATCH * HIST
NUM_CORES = 2
NUM_SUBCORES = 16
NW = NUM_CORES * NUM_SUBCORES
BPW = BATCH // NW
PER_W = BPW * HIST
IDX_ALL = PER_W // 128
NLIST = BPW // 128
NK = BPW // 16

_mesh = plsc.VectorSubcoreMesh(core_axis_name="c", subcore_axis_name="s")


@functools.partial(
    pl.kernel,
    mesh=_mesh,
    out_type=jax.ShapeDtypeStruct((HIST, DIM, BATCH), jnp.float32),
    scratch_types=[
        pltpu.VMEM((IDX_ALL, 128), jnp.int32),
        pltpu.VMEM((2, NLIST, 128), jnp.int32),
        pltpu.VMEM((2, BPW, DIM), jnp.float32),
        pltpu.VMEM((2, DIM, BPW), jnp.float32),
        pltpu.SemaphoreType.DMA,
        pltpu.SemaphoreType.DMA,
        pltpu.SemaphoreType.DMA,
        pltpu.SemaphoreType.DMA,
    ],
    compiler_params=pltpu.CompilerParams(
        use_tc_tiling_on_sc=False, needs_layout_passes=False
    ),
)
def _gather_kernel(idx_hbm, table_hbm, out_hbm, idx_v, list_v, rows_v, tbuf_v,
                   sg0, sg1, so0, so1):
    wid = lax.axis_index("s") * NUM_CORES + lax.axis_index("c")
    wb0 = pl.multiple_of(wid * BPW, BPW)
    idx_row0 = pl.multiple_of(wid * IDX_ALL, 8)
    lanes = lax.iota(jnp.int32, 16)

    pltpu.sync_copy(idx_hbm.at[pl.ds(idx_row0, IDX_ALL)], idx_v)

    def build_list(l, buf):
        for k in range(NK):
            p = (k * 16 + lanes) * HIST + l
            v = plsc.load_gather(idx_v, [p >> 7, p & 127])
            list_v[buf, k // 8, pl.ds((k % 8) * 16, 16)] = v

    def gather_copies(buf, sem):
        return [
            pltpu.make_async_copy(
                table_hbm.at[list_v.at[buf].at[j]],
                rows_v.at[buf].at[pl.ds(j * 128, 128)],
                sem,
            )
            for j in range(NLIST)
        ]

    def fire_gathers(buf, sem):
        for c in gather_copies(buf, sem):
            c.start()

    def wait_gathers(buf, sem):
        for c in gather_copies(buf, sem):
            c.wait()

    def transpose(buf):
        rows_ref = rows_v.at[buf]

        def dbody(d, carry):
            dvec = jnp.full((16,), d, jnp.int32)
            for k in range(NK):
                pos = k * 16 + lanes
                v = plsc.load_gather(rows_ref, [pos, dvec])
                tbuf_v[buf, d, pl.ds(k * 16, 16)] = v
            return carry

        lax.fori_loop(0, DIM, dbody, 0)

    def out_copy(l, buf, sem):
        return pltpu.make_async_copy(
            tbuf_v.at[buf],
            out_hbm.at[l].at[:, pl.ds(wb0, BPW)],
            sem,
        )

    build_list(0, 0)
    fire_gathers(0, sg0)

    def body(i, carry):
        l0 = 2 * i
        l1 = 2 * i + 1

        build_list(l1, 1)
        wait_gathers(0, sg0)
        fire_gathers(1, sg1)

        @pl.when(i > 0)
        def _():
            out_copy(l0 - 2, 0, so0).wait()

        transpose(0)
        out_copy(l0, 0, so0).start()

        @pl.when(i < HIST // 2 - 1)
        def _():
            build_list(l0 + 2, 0)

        wait_gathers(1, sg1)

        @pl.when(i < HIST // 2 - 1)
        def _():
            fire_gathers(0, sg0)

        @pl.when(i > 0)
        def _():
            out_copy(l1 - 2, 1, so1).wait()

        transpose(1)
        out_copy(l1, 1, so1).start()
        return carry

    lax.fori_loop(0, HIST // 2, body, 0)
    out_copy(HIST - 2, 0, so0).wait()
    out_copy(HIST - 1, 1, so1).wait()


def kernel(input, weight):
    idx = input.astype(jnp.int32).reshape(TOTAL // 128, 128)
    out = _gather_kernel(idx, weight)
    return jnp.transpose(out, (2, 0, 1))


# speedup vs baseline: 2.2296x; 1.4946x over previous
import functools

import jax
import jax.numpy as jnp
from jax import lax
from jax.experimental import pallas as pl
from jax.experimental.pallas import tpu as pltpu
from jax.experimental.pallas import tpu_sc as plsc

BATCH = 16384
HIST = 50
DIM = 32
TOTAL = BATCH * HIST
NUM_CORES = 2
NUM_SUBCORES = 16
NW = NUM_CORES * NUM_SUBCORES
BPW = BATCH // NW
PER_W = BPW * HIST
IDX_ALL = PER_W // 128
NLIST = BPW // 128
NK = BPW // 16

_mesh = plsc.VectorSubcoreMesh(core_axis_name="c", subcore_axis_name="s")


@functools.partial(
    pl.kernel,
    mesh=_mesh,
    out_type=jax.ShapeDtypeStruct((HIST, DIM, BATCH), jnp.float32),
    scratch_types=[
        pltpu.VMEM((IDX_ALL, 128), jnp.int32),
        pltpu.VMEM((2, NLIST, 128), jnp.int32),
        pltpu.VMEM((2, BPW, DIM), jnp.float32),
        pltpu.VMEM((2, DIM, BPW), jnp.float32),
        pltpu.SemaphoreType.DMA,
        pltpu.SemaphoreType.DMA,
        pltpu.SemaphoreType.DMA,
        pltpu.SemaphoreType.DMA,
    ],
    compiler_params=pltpu.CompilerParams(
        use_tc_tiling_on_sc=False, needs_layout_passes=False
    ),
)
def _gather_kernel(idx_hbm, table_hbm, out_hbm, idx_v, list_v, rows_v, tbuf_v,
                   sg0, sg1, so0, so1):
    wid = lax.axis_index("s") * NUM_CORES + lax.axis_index("c")
    wb0 = pl.multiple_of(wid * BPW, BPW)
    idx_row0 = pl.multiple_of(wid * IDX_ALL, 8)
    lanes = lax.iota(jnp.int32, 16)

    pltpu.sync_copy(idx_hbm.at[pl.ds(idx_row0, IDX_ALL)], idx_v)

    def build_list(l, buf):
        for k in range(NK):
            p = (k * 16 + lanes) * HIST + l
            v = plsc.load_gather(idx_v, [p >> 7, p & 127])
            list_v[buf, k // 8, pl.ds((k % 8) * 16, 16)] = v

    def gather_copies(buf, sem):
        return [
            pltpu.make_async_copy(
                table_hbm.at[list_v.at[buf].at[j]],
                rows_v.at[buf].at[pl.ds(j * 128, 128)],
                sem,
            )
            for j in range(NLIST)
        ]

    def fire_gathers(buf, sem):
        for c in gather_copies(buf, sem):
            c.start()

    def wait_gathers(buf, sem):
        for c in gather_copies(buf, sem):
            c.wait()

    colvecs = [
        ((j + lanes) & 15) + h * 16 for h in range(DIM // 16) for j in range(16)
    ]

    def transpose(buf):
        rows_ref = rows_v.at[buf]
        tref = tbuf_v.at[buf]

        def btbody(bt, carry):
            rowvec = bt * 16 + lanes
            for cv in colvecs:
                v = plsc.load_gather(rows_ref, [rowvec, cv])
                plsc.store_scatter(tref, [cv, rowvec], v)
            return carry

        lax.fori_loop(0, NK, btbody, 0)

    def out_copy(l, buf, sem):
        return pltpu.make_async_copy(
            tbuf_v.at[buf],
            out_hbm.at[l].at[:, pl.ds(wb0, BPW)],
            sem,
        )

    build_list(0, 0)
    fire_gathers(0, sg0)

    def body(i, carry):
        l0 = 2 * i
        l1 = 2 * i + 1

        build_list(l1, 1)
        wait_gathers(0, sg0)
        fire_gathers(1, sg1)

        @pl.when(i > 0)
        def _():
            out_copy(l0 - 2, 0, so0).wait()

        transpose(0)
        out_copy(l0, 0, so0).start()

        @pl.when(i < HIST // 2 - 1)
        def _():
            build_list(l0 + 2, 0)

        wait_gathers(1, sg1)

        @pl.when(i < HIST // 2 - 1)
        def _():
            fire_gathers(0, sg0)

        @pl.when(i > 0)
        def _():
            out_copy(l1 - 2, 1, so1).wait()

        transpose(1)
        out_copy(l1, 1, so1).start()
        return carry

    lax.fori_loop(0, HIST // 2, body, 0)
    out_copy(HIST - 2, 0, so0).wait()
    out_copy(HIST - 1, 1, so1).wait()


def kernel(input, weight):
    idx = input.astype(jnp.int32).reshape(TOTAL // 128, 128)
    out = _gather_kernel(idx, weight)
    return jnp.transpose(out, (2, 0, 1))


# speedup vs baseline: 2.2730x; 1.0195x over previous
import functools

import jax
import jax.numpy as jnp
from jax import lax
from jax.experimental import pallas as pl
from jax.experimental.pallas import tpu as pltpu
from jax.experimental.pallas import tpu_sc as plsc

BATCH = 16384
HIST = 50
DIM = 32
NUM_CORES = 2
NUM_SUBCORES = 16
NW = NUM_CORES * NUM_SUBCORES
BPW = BATCH // NW
NLIST = BPW // 128
NK = BPW // 16

_mesh = plsc.VectorSubcoreMesh(core_axis_name="c", subcore_axis_name="s")


@functools.partial(
    pl.kernel,
    mesh=_mesh,
    out_type=jax.ShapeDtypeStruct((HIST, DIM, BATCH), jnp.float32),
    scratch_types=[
        pltpu.VMEM((HIST, BPW), jnp.int32),
        pltpu.VMEM((2, BPW, DIM), jnp.float32),
        pltpu.VMEM((2, DIM, BPW), jnp.float32),
        pltpu.SemaphoreType.DMA,
        pltpu.SemaphoreType.DMA,
        pltpu.SemaphoreType.DMA,
        pltpu.SemaphoreType.DMA,
    ],
    compiler_params=pltpu.CompilerParams(
        use_tc_tiling_on_sc=False, needs_layout_passes=False
    ),
)
def _gather_kernel(idx_hbm, table_hbm, out_hbm, idx_v, rows_v, tbuf_v,
                   sg0, sg1, so0, so1):
    wid = lax.axis_index("s") * NUM_CORES + lax.axis_index("c")
    wb0 = pl.multiple_of(wid * BPW, BPW)
    lanes = lax.iota(jnp.int32, 16)

    pltpu.sync_copy(idx_hbm.at[:, pl.ds(wb0, BPW)], idx_v)

    def gather_copies(l, buf, sem):
        return [
            pltpu.make_async_copy(
                table_hbm.at[idx_v.at[l].at[pl.ds(j * 128, 128)]],
                rows_v.at[buf].at[pl.ds(j * 128, 128)],
                sem,
            )
            for j in range(NLIST)
        ]

    def fire_gathers(l, buf, sem):
        for c in gather_copies(l, buf, sem):
            c.start()

    def wait_gathers(l, buf, sem):
        for c in gather_copies(l, buf, sem):
            c.wait()

    colvecs = [
        ((j + lanes) & 15) + h * 16 for h in range(DIM // 16) for j in range(16)
    ]

    def transpose(buf):
        rows_ref = rows_v.at[buf]
        tref = tbuf_v.at[buf]

        def btbody(bt, carry):
            rowvec = bt * 16 + lanes
            for cv in colvecs:
                v = plsc.load_gather(rows_ref, [rowvec, cv])
                plsc.store_scatter(tref, [cv, rowvec], v)
            return carry

        lax.fori_loop(0, NK, btbody, 0)

    def out_copy(l, buf, sem):
        return pltpu.make_async_copy(
            tbuf_v.at[buf],
            out_hbm.at[l].at[:, pl.ds(wb0, BPW)],
            sem,
        )

    fire_gathers(0, 0, sg0)

    def body(i, carry):
        l0 = 2 * i
        l1 = 2 * i + 1

        wait_gathers(l0, 0, sg0)
        fire_gathers(l1, 1, sg1)

        @pl.when(i > 0)
        def _():
            out_copy(l0 - 2, 0, so0).wait()

        transpose(0)
        out_copy(l0, 0, so0).start()

        wait_gathers(l1, 1, sg1)

        @pl.when(i < HIST // 2 - 1)
        def _():
            fire_gathers(l0 + 2, 0, sg0)

        @pl.when(i > 0)
        def _():
            out_copy(l1 - 2, 1, so1).wait()

        transpose(1)
        out_copy(l1, 1, so1).start()
        return carry

    lax.fori_loop(0, HIST // 2, body, 0)
    out_copy(HIST - 2, 0, so0).wait()
    out_copy(HIST - 1, 1, so1).wait()


def kernel(input, weight):
    idx_t = input.T.astype(jnp.int32)
    out = _gather_kernel(idx_t, weight)
    return jnp.transpose(out, (2, 0, 1))


# speedup vs baseline: 2.4976x; 1.0988x over previous
import functools

import jax
import jax.numpy as jnp
from jax import lax
from jax.experimental import pallas as pl
from jax.experimental.pallas import tpu as pltpu
from jax.experimental.pallas import tpu_sc as plsc

BATCH = 16384
HIST = 50
DIM = 32
NUM_CORES = 2
NUM_SUBCORES = 16
NW = NUM_CORES * NUM_SUBCORES
BPW = BATCH // NW
NLIST = BPW // 128
NK = BPW // 16

_mesh = plsc.VectorSubcoreMesh(core_axis_name="c", subcore_axis_name="s")


@functools.partial(
    pl.kernel,
    mesh=_mesh,
    out_type=jax.ShapeDtypeStruct((HIST, DIM, BATCH), jnp.float32),
    scratch_types=[
        pltpu.VMEM((HIST, BPW), jnp.int32),
        pltpu.VMEM((2, BPW, DIM), jnp.float32),
        pltpu.VMEM((2, DIM, BPW), jnp.float32),
        pltpu.SemaphoreType.DMA,
        pltpu.SemaphoreType.DMA,
        pltpu.SemaphoreType.DMA,
        pltpu.SemaphoreType.DMA,
    ],
    compiler_params=pltpu.CompilerParams(
        use_tc_tiling_on_sc=False, needs_layout_passes=False
    ),
)
def _gather_kernel(idx_hbm, table_hbm, out_hbm, idx_v, rows_v, tbuf_v,
                   sg0, sg1, so0, so1):
    wid = lax.axis_index("s") * NUM_CORES + lax.axis_index("c")
    wb0 = pl.multiple_of(wid * BPW, BPW)
    lanes = lax.iota(jnp.int32, 16)

    pltpu.sync_copy(idx_hbm.at[:, pl.ds(wb0, BPW)], idx_v)

    def gather_copies(l, buf, sem):
        return [
            pltpu.make_async_copy(
                table_hbm.at[idx_v.at[l].at[pl.ds(j * 128, 128)]],
                rows_v.at[buf].at[pl.ds(j * 128, 128)],
                sem,
            )
            for j in range(NLIST)
        ]

    def fire_gathers(l, buf, sem):
        for c in gather_copies(l, buf, sem):
            c.start()

    def wait_gathers(l, buf, sem):
        for c in gather_copies(l, buf, sem):
            c.wait()

    colvecs = [
        ((j + lanes) & 15) + h * 16 for h in range(DIM // 16) for j in range(16)
    ]

    def transpose(buf):
        rows_ref = rows_v.at[buf]
        tref = tbuf_v.at[buf]

        def btbody(bt, carry):
            rv0 = bt * 32 + lanes
            rv1 = rv0 + 16
            for cv in colvecs:
                v0 = plsc.load_gather(rows_ref, [rv0, cv])
                v1 = plsc.load_gather(rows_ref, [rv1, cv])
                plsc.store_scatter(tref, [cv, rv0], v0)
                plsc.store_scatter(tref, [cv, rv1], v1)
            return carry

        lax.fori_loop(0, NK // 2, btbody, 0)

    def out_copy(l, buf, sem):
        return pltpu.make_async_copy(
            tbuf_v.at[buf],
            out_hbm.at[l].at[:, pl.ds(wb0, BPW)],
            sem,
        )

    fire_gathers(0, 0, sg0)

    def body(i, carry):
        l0 = 2 * i
        l1 = 2 * i + 1

        wait_gathers(l0, 0, sg0)
        fire_gathers(l1, 1, sg1)

        @pl.when(i > 0)
        def _():
            out_copy(l0 - 2, 0, so0).wait()

        transpose(0)
        out_copy(l0, 0, so0).start()

        wait_gathers(l1, 1, sg1)

        @pl.when(i < HIST // 2 - 1)
        def _():
            fire_gathers(l0 + 2, 0, sg0)

        @pl.when(i > 0)
        def _():
            out_copy(l1 - 2, 1, so1).wait()

        transpose(1)
        out_copy(l1, 1, so1).start()
        return carry

    lax.fori_loop(0, HIST // 2, body, 0)
    out_copy(HIST - 2, 0, so0).wait()
    out_copy(HIST - 1, 1, so1).wait()


def kernel(input, weight):
    idx_t = input.T.astype(jnp.int32)
    out = _gather_kernel(idx_t, weight)
    return jnp.transpose(out, (2, 0, 1))


# speedup vs baseline: 2.5231x; 1.0102x over previous
import functools

import jax
import jax.numpy as jnp
from jax import lax
from jax.experimental import pallas as pl
from jax.experimental.pallas import tpu as pltpu
from jax.experimental.pallas import tpu_sc as plsc

BATCH = 16384
HIST = 50
DIM = 32
NUM_CORES = 2
NUM_SUBCORES = 16
NW = NUM_CORES * NUM_SUBCORES
BPW = BATCH // NW
NLIST = BPW // 128
NK = BPW // 16

_mesh = plsc.VectorSubcoreMesh(core_axis_name="c", subcore_axis_name="s")


@functools.partial(
    pl.kernel,
    mesh=_mesh,
    out_type=jax.ShapeDtypeStruct((HIST, DIM, BATCH), jnp.float32),
    scratch_types=[
        pltpu.VMEM((HIST, BPW), jnp.int32),
        pltpu.VMEM((2, BPW, DIM), jnp.float32),
        pltpu.VMEM((2, DIM, BPW), jnp.float32),
        pltpu.SemaphoreType.DMA,
        pltpu.SemaphoreType.DMA,
        pltpu.SemaphoreType.DMA,
        pltpu.SemaphoreType.DMA,
    ],
    compiler_params=pltpu.CompilerParams(
        use_tc_tiling_on_sc=False, needs_layout_passes=False
    ),
)
def _gather_kernel(idx_hbm, table_hbm, out_hbm, idx_v, rows_v, tbuf_v,
                   sg0, sg1, so0, so1):
    wid = lax.axis_index("s") * NUM_CORES + lax.axis_index("c")
    wb0 = pl.multiple_of(wid * BPW, BPW)
    lanes = lax.iota(jnp.int32, 16)

    pltpu.sync_copy(idx_hbm.at[:, pl.ds(wb0, BPW)], idx_v)

    def gather_copies(l, buf, sem):
        return [
            pltpu.make_async_copy(
                table_hbm.at[idx_v.at[l].at[pl.ds(j * 128, 128)]],
                rows_v.at[buf].at[pl.ds(j * 128, 128)],
                sem,
            )
            for j in range(NLIST)
        ]

    def fire_gathers(l, buf, sem):
        for c in gather_copies(l, buf, sem):
            c.start()

    def wait_gathers(l, buf, sem):
        for c in gather_copies(l, buf, sem):
            c.wait()

    colvecs = [
        ((j + lanes) & 15) + h * 16 for h in range(DIM // 16) for j in range(16)
    ]

    def transpose(buf):
        rows_ref = rows_v.at[buf]
        tref = tbuf_v.at[buf]

        def btbody(bt, carry):
            rvs = [bt * 64 + u * 16 + lanes for u in range(4)]
            for cv in colvecs:
                vs = [plsc.load_gather(rows_ref, [rv, cv]) for rv in rvs]
                for rv, v in zip(rvs, vs):
                    plsc.store_scatter(tref, [cv, rv], v)
            return carry

        lax.fori_loop(0, NK // 4, btbody, 0)

    def out_copy(l, buf, sem):
        return pltpu.make_async_copy(
            tbuf_v.at[buf],
            out_hbm.at[l].at[:, pl.ds(wb0, BPW)],
            sem,
        )

    fire_gathers(0, 0, sg0)

    def body(i, carry):
        l0 = 2 * i
        l1 = 2 * i + 1

        wait_gathers(l0, 0, sg0)
        fire_gathers(l1, 1, sg1)

        @pl.when(i > 0)
        def _():
            out_copy(l0 - 2, 0, so0).wait()

        transpose(0)
        out_copy(l0, 0, so0).start()

        wait_gathers(l1, 1, sg1)

        @pl.when(i < HIST // 2 - 1)
        def _():
            fire_gathers(l0 + 2, 0, sg0)

        @pl.when(i > 0)
        def _():
            out_copy(l1 - 2, 1, so1).wait()

        transpose(1)
        out_copy(l1, 1, so1).start()
        return carry

    lax.fori_loop(0, HIST // 2, body, 0)
    out_copy(HIST - 2, 0, so0).wait()
    out_copy(HIST - 1, 1, so1).wait()


def kernel(input, weight):
    idx_t = input.T.astype(jnp.int32)
    out = _gather_kernel(idx_t, weight)
    return jnp.transpose(out, (2, 0, 1))
